# baseline (speedup 1.0000x reference)
import functools

import jax
import jax.numpy as jnp
from jax.experimental import pallas as pl
import jax.experimental.pallas.tpu as pltpu

N, F_IN, H, C = 10000, 128, 128, 64
BM = 400


def _layer1_kernel(adj_ref, x_ref, w1_ref, b1_ref, out_ref, s1_ref):
    @pl.when(pl.program_id(0) == 0)
    def _():
        s1_ref[...] = jnp.dot(x_ref[...], w1_ref[...],
                              preferred_element_type=jnp.float32)

    acc = jnp.dot(adj_ref[...], s1_ref[...],
                  preferred_element_type=jnp.float32)
    out_ref[...] = jnp.maximum(acc + b1_ref[...], 0.0)


def _layer2_kernel(adj_ref, h1_ref, w2_ref, b2_ref, out_ref, t_ref):
    @pl.when(pl.program_id(0) == 0)
    def _():
        t_ref[...] = jnp.dot(h1_ref[...], w2_ref[...],
                             preferred_element_type=jnp.float32)

    z = jnp.dot(adj_ref[...], t_ref[...],
                preferred_element_type=jnp.float32) + b2_ref[...]
    m = jnp.max(z, axis=1, keepdims=True)
    zs = z - m
    lse = jnp.log(jnp.sum(jnp.exp(zs), axis=1, keepdims=True))
    out_ref[...] = zs - lse


@functools.partial(jax.jit)
def kernel(x, adj, W1, b1, W2, b2):
    grid = (N // BM,)
    adj_spec = pl.BlockSpec((BM, N), lambda i: (i, 0))
    full = lambda shape: pl.BlockSpec(shape, lambda i: (0, 0))

    h1 = pl.pallas_call(
        _layer1_kernel,
        grid=grid,
        in_specs=[adj_spec, full((N, F_IN)), full((F_IN, H)), full((1, H))],
        out_specs=pl.BlockSpec((BM, H), lambda i: (i, 0)),
        out_shape=jax.ShapeDtypeStruct((N, H), jnp.float32),
        scratch_shapes=[pltpu.VMEM((N, H), jnp.float32)],
    )(adj, x, W1, b1.reshape(1, H))

    out = pl.pallas_call(
        _layer2_kernel,
        grid=grid,
        in_specs=[adj_spec, full((N, H)), full((H, C)), full((1, C))],
        out_specs=pl.BlockSpec((BM, C), lambda i: (i, 0)),
        out_shape=jax.ShapeDtypeStruct((N, C), jnp.float32),
        scratch_shapes=[pltpu.VMEM((N, C), jnp.float32)],
    )(adj, h1, W2, b2.reshape(1, C))

    return out
